# baseline (speedup 1.0000x reference)
import functools

import jax
import jax.numpy as jnp
from jax import lax
from jax.experimental import pallas as pl
from jax.experimental.pallas import tpu as pltpu
from jax.experimental.pallas import tpu_sc as plsc

_B = 4096
_S = 200
_D = 64
_NW = 32
_DPW = _D // _NW
_C = 256
_V = _C // 16

_X = 2816
_LSC = _B - _X
_BLKL = 256


def _tc_body(x_ref, o_ref):
    o_ref[...] = jnp.sum(x_ref[...], axis=0)


def _tc_sum(x3):
    return pl.pallas_call(
        _tc_body,
        grid=(_X // _BLKL,),
        in_specs=[pl.BlockSpec((_S, _D, _BLKL), lambda i: (0, 0, i))],
        out_specs=pl.BlockSpec((_D, _BLKL), lambda i: (0, i)),
        out_shape=jax.ShapeDtypeStruct((_D, _X), jnp.float32),
    )(x3)


def _sc_body(x_hbm, out_hbm, buf0, buf1, stage, sem0, sem1):
    cid = lax.axis_index("c")
    sid = lax.axis_index("s")
    w = sid * 2 + cid

    bufs = (buf0, buf1)
    sems = (sem0, sem1)

    def start(j, slot):
        dd = lax.rem(j, _DPW)
        c = lax.div(j, _DPW)
        d = w * _DPW + dd
        return pltpu.async_copy(
            x_hbm.at[:, pl.ds(d, 1), pl.ds(_X + c * _C, _C)],
            bufs[slot],
            sems[slot],
        )

    def wait(slot):
        pltpu.make_async_copy(
            x_hbm.at[:, pl.ds(0, 1), pl.ds(0, _C)], bufs[slot], sems[slot]
        ).wait()

    def compute(j, slot):
        buf = bufs[slot]
        dd = lax.rem(j, _DPW)
        c = lax.div(j, _DPW)
        d = w * _DPW + dd
        acc = tuple(buf[0, 0, pl.ds(i * 16, 16)] for i in range(_V))

        def body(s, acc):
            return tuple(
                acc[i] + buf[s, 0, pl.ds(i * 16, 16)] for i in range(_V)
            )

        acc = lax.fori_loop(1, _S, body, acc)
        for i in range(_V):
            stage[0, pl.ds(i * 16, 16)] = acc[i]
        pltpu.sync_copy(stage, out_hbm.at[pl.ds(d, 1), pl.ds(c * _C, _C)])

    njobs = _DPW * (_LSC // _C)
    start(0, 0)
    start(1, 1)

    def outer(j2, _):
        j = j2 * 2
        wait(0)
        compute(j, 0)
        start(j + 2, 0)
        wait(1)
        compute(j + 1, 1)
        start(j + 3, 1)
        return _

    lax.fori_loop(0, njobs // 2 - 1, outer, 0)
    wait(0)
    compute(njobs - 2, 0)
    wait(1)
    compute(njobs - 1, 1)


def _sc_sum(x3):
    mesh = plsc.VectorSubcoreMesh(core_axis_name="c", subcore_axis_name="s")
    f = functools.partial(
        pl.kernel,
        mesh=mesh,
        out_type=jax.ShapeDtypeStruct((_D, _LSC), jnp.float32),
        scratch_types=[
            pltpu.VMEM((_S, 1, _C), jnp.float32),
            pltpu.VMEM((_S, 1, _C), jnp.float32),
            pltpu.VMEM((1, _C), jnp.float32),
            pltpu.SemaphoreType.DMA,
            pltpu.SemaphoreType.DMA,
        ],
    )(_sc_body)
    return f(x3)


def kernel(inputs):
    x3 = jnp.transpose(inputs, (1, 2, 0))
    out_sc = _sc_sum(x3)
    out_tc = _tc_sum(x3)
    out_t = jnp.concatenate([out_tc, out_sc], axis=1)
    return jnp.transpose(out_t, (1, 0))
